# baseline (speedup 1.0000x reference)
import jax
import jax.numpy as jnp
from jax.experimental import pallas as pl
from jax.experimental.pallas import tpu as pltpu

_NCLS = 512
_NCHUNK = 16


def _chunk(x, pa, o_buf, rows):
    xx = jnp.sum(x * x, axis=1, keepdims=True)
    xa = jnp.concatenate([x * -2.0, xx, jnp.ones_like(xx)], axis=1)
    dn = (((1,), (1,)), ((), ()))
    d2 = jax.lax.dot_general(xa, pa, dn, preferred_element_type=jnp.float32)
    m = jnp.where(d2[:, :_NCLS] < d2[:, _NCLS:], d2[:, :_NCLS], d2[:, _NCLS:])
    ab = jnp.abs(m) + 1e-30
    o_buf[rows, :] = ab * jax.lax.rsqrt(ab)


def _glvq_body(x_ref, p_ref, o_hbm, buf, *sems):
    p = p_ref[:]
    pp = jnp.sum(p * p, axis=1, keepdims=True)
    pa = jnp.concatenate([p, jnp.ones_like(pp), pp], axis=1)
    chunk = x_ref.shape[0] // _NCHUNK

    copies = []
    for i in range(_NCHUNK):
        rows = pl.ds(i * chunk, chunk)
        _chunk(x_ref[rows, :], pa, buf, rows)
        cp = pltpu.make_async_copy(buf.at[rows, :], o_hbm.at[rows, :], sems[i])
        cp.start()
        copies.append(cp)
    for cp in copies:
        cp.wait()


def kernel(x, protos):
    batch = x.shape[0]
    return pl.pallas_call(
        _glvq_body,
        out_shape=jax.ShapeDtypeStruct((batch, _NCLS), jnp.float32),
        out_specs=pl.BlockSpec(memory_space=pltpu.MemorySpace.HBM),
        scratch_shapes=[pltpu.VMEM((batch, _NCLS), jnp.float32)]
        + [pltpu.SemaphoreType.DMA] * _NCHUNK,
    )(x, protos)


# speedup vs baseline: 1.1189x; 1.1189x over previous
import jax
import jax.numpy as jnp
from jax.experimental import pallas as pl
from jax.experimental.pallas import tpu as pltpu

_NCLS = 512
_NCHUNK = 8


def _chunk(x, pa, o_buf, rows):
    xx = jnp.sum(x * x, axis=1, keepdims=True)
    xa = jnp.concatenate([x * -2.0, xx, jnp.ones_like(xx)], axis=1)
    dn = (((1,), (1,)), ((), ()))
    d2 = jax.lax.dot_general(xa, pa, dn, preferred_element_type=jnp.float32)
    m = jnp.where(d2[:, :_NCLS] < d2[:, _NCLS:], d2[:, :_NCLS], d2[:, _NCLS:])
    ab = jnp.abs(m) + 1e-30
    o_buf[rows, :] = ab * jax.lax.rsqrt(ab)


def _glvq_body(x_ref, p_ref, o_hbm, buf, *sems):
    p = p_ref[:]
    pp = jnp.sum(p * p, axis=1, keepdims=True)
    pa = jnp.concatenate([p, jnp.ones_like(pp), pp], axis=1)
    chunk = x_ref.shape[0] // _NCHUNK

    copies = []
    for i in range(_NCHUNK):
        rows = pl.ds(i * chunk, chunk)
        _chunk(x_ref[rows, :], pa, buf, rows)
        cp = pltpu.make_async_copy(buf.at[rows, :], o_hbm.at[rows, :], sems[i])
        cp.start()
        copies.append(cp)
    for cp in copies:
        cp.wait()


def kernel(x, protos):
    batch = x.shape[0]
    return pl.pallas_call(
        _glvq_body,
        out_shape=jax.ShapeDtypeStruct((batch, _NCLS), jnp.float32),
        out_specs=pl.BlockSpec(memory_space=pltpu.MemorySpace.HBM),
        scratch_shapes=[pltpu.VMEM((batch, _NCLS), jnp.float32)]
        + [pltpu.SemaphoreType.DMA] * _NCHUNK,
    )(x, protos)


# speedup vs baseline: 1.1192x; 1.0002x over previous
import jax
import jax.numpy as jnp
from jax.experimental import pallas as pl
from jax.experimental.pallas import tpu as pltpu

_NCLS = 512
_NCHUNK = 8


def _chunk(x, pa, o_buf, rows):
    xx = jnp.sum(x * x, axis=1, keepdims=True)
    xa = jnp.concatenate([x * -2.0, xx, jnp.ones_like(xx)], axis=1)
    dn = (((1,), (1,)), ((), ()))
    d2 = jax.lax.dot_general(xa.astype(jnp.bfloat16), pa, dn,
                             preferred_element_type=jnp.float32)
    m = jnp.where(d2[:, :_NCLS] < d2[:, _NCLS:], d2[:, :_NCLS], d2[:, _NCLS:])
    ab = jnp.abs(m) + 1e-30
    o_buf[rows, :] = ab * jax.lax.rsqrt(ab)


def _glvq_body(x_ref, p_ref, o_hbm, buf, *sems):
    p = p_ref[:]
    pp = jnp.sum(p * p, axis=1, keepdims=True)
    pa = jnp.concatenate([p, jnp.ones_like(pp), pp], axis=1).astype(jnp.bfloat16)
    chunk = x_ref.shape[0] // _NCHUNK

    copies = []
    for i in range(_NCHUNK):
        rows = pl.ds(i * chunk, chunk)
        _chunk(x_ref[rows, :], pa, buf, rows)
        cp = pltpu.make_async_copy(buf.at[rows, :], o_hbm.at[rows, :], sems[i])
        cp.start()
        copies.append(cp)
    for cp in copies:
        cp.wait()


def kernel(x, protos):
    batch = x.shape[0]
    return pl.pallas_call(
        _glvq_body,
        out_shape=jax.ShapeDtypeStruct((batch, _NCLS), jnp.float32),
        out_specs=pl.BlockSpec(memory_space=pltpu.MemorySpace.HBM),
        scratch_shapes=[pltpu.VMEM((batch, _NCLS), jnp.float32)]
        + [pltpu.SemaphoreType.DMA] * _NCHUNK,
    )(x, protos)
